# baseline (speedup 1.0000x reference)
import functools

import jax
import jax.numpy as jnp
from jax import lax
from jax.experimental import pallas as pl
from jax.experimental.pallas import tpu as pltpu
from jax.experimental.pallas import tpu_sc as plsc

_NC = 2
_NS = 16
_NW = _NC * _NS
_CHUNK = 128


def _make_sc_gather(B, D):
    assert B % (_NW * _CHUNK) == 0
    b_per_w = B // _NW
    chunks = b_per_w // _CHUNK
    mesh = plsc.VectorSubcoreMesh(core_axis_name="c", subcore_axis_name="s")

    @functools.partial(
        pl.kernel,
        mesh=mesh,
        out_type=[
            jax.ShapeDtypeStruct((B, D), jnp.float32),
            jax.ShapeDtypeStruct((B, D), jnp.float32),
        ],
        scratch_types=[
            pltpu.VMEM((chunks, _CHUNK), jnp.int32),
            pltpu.VMEM((chunks, _CHUNK), jnp.int32),
            pltpu.VMEM((b_per_w, D), jnp.float32),
            pltpu.VMEM((b_per_w, D), jnp.float32),
            pltpu.SemaphoreType.DMA,
        ],
        compiler_params=pltpu.CompilerParams(use_tc_tiling_on_sc=False),
    )
    def sc_gather(uidx_hbm, midx_hbm, utab_hbm, mtab_hbm, u_out, m_out,
                  uidx_v, midx_v, urows_v, mrows_v, sem):
        wid = lax.axis_index("s") * _NC + lax.axis_index("c")
        row0 = wid * chunks
        base = wid * b_per_w
        pltpu.sync_copy(uidx_hbm.at[pl.ds(row0, chunks)], uidx_v)
        pltpu.sync_copy(midx_hbm.at[pl.ds(row0, chunks)], midx_v)
        copies = []
        for j in range(chunks):
            copies.append(pltpu.async_copy(
                utab_hbm.at[uidx_v.at[j]],
                urows_v.at[pl.ds(j * _CHUNK, _CHUNK)], sem))
            copies.append(pltpu.async_copy(
                mtab_hbm.at[midx_v.at[j]],
                mrows_v.at[pl.ds(j * _CHUNK, _CHUNK)], sem))
        for c in copies:
            c.wait()
        pltpu.sync_copy(urows_v, u_out.at[pl.ds(base, b_per_w)])
        pltpu.sync_copy(mrows_v, m_out.at[pl.ds(base, b_per_w)])

    return sc_gather


def _dense_body(u_ref, m_ref, wu_ref, bu_ref, wm_ref, bm_ref, o_ref):
    du = jnp.dot(u_ref[...], wu_ref[...],
                 preferred_element_type=jnp.float32) + bu_ref[...]
    dm = jnp.dot(m_ref[...], wm_ref[...],
                 preferred_element_type=jnp.float32) + bm_ref[...]
    o_ref[...] = jnp.sum(du * dm, axis=1, keepdims=True)


def kernel(user_input, movie_input, user_emb, item_emb, Wu, bu, Wm, bm):
    B = user_input.shape[0]
    K = user_emb.shape[1]
    H = Wu.shape[1]

    uidx = user_input.reshape(B // _CHUNK, _CHUNK)
    midx = movie_input.reshape(B // _CHUNK, _CHUNK)
    u_rows, m_rows = _make_sc_gather(B, K)(uidx, midx, user_emb, item_emb)

    BLK = 2048
    z = pl.pallas_call(
        _dense_body,
        grid=(B // BLK,),
        in_specs=[
            pl.BlockSpec((BLK, K), lambda i: (i, 0)),
            pl.BlockSpec((BLK, K), lambda i: (i, 0)),
            pl.BlockSpec((K, H), lambda i: (0, 0)),
            pl.BlockSpec((1, H), lambda i: (0, 0)),
            pl.BlockSpec((K, H), lambda i: (0, 0)),
            pl.BlockSpec((1, H), lambda i: (0, 0)),
        ],
        out_specs=pl.BlockSpec((BLK, 1), lambda i: (i, 0)),
        out_shape=jax.ShapeDtypeStruct((B, 1), jnp.float32),
    )(u_rows, m_rows, Wu, bu.reshape(1, H), Wm, bm.reshape(1, H))
    return z
